# baseline (speedup 1.0000x reference)
import functools

import jax
import jax.numpy as jnp
from jax import lax
from jax.experimental import pallas as pl
from jax.experimental.pallas import tpu as pltpu
from jax.experimental.pallas import tpu_sc as plsc

N_ROWS = 1000000
D = 64
PAD_H = 56
PAD_D = 128
NC = 2
NS = 16
NW = NC * NS
NBUF = 8
K = 4


def _make_embed_kernel(batch: int, hist: int):
    rows_per_w = batch // NW
    assert batch % NW == 0 and rows_per_w % NBUF == 0

    mesh = plsc.VectorSubcoreMesh(
        core_axis_name="c", subcore_axis_name="s",
        num_cores=NC, num_subcores=NS,
    )

    @functools.partial(
        pl.kernel,
        out_type=jax.ShapeDtypeStruct((batch, PAD_H, PAD_D), jnp.float32),
        mesh=mesh,
        scratch_types=(
            pltpu.VMEM((rows_per_w, PAD_H), jnp.int32),
            [pltpu.VMEM((PAD_H, D), jnp.float32) for _ in range(NBUF)],
            [pltpu.SemaphoreType.DMA for _ in range(NBUF)],
            [pltpu.SemaphoreType.DMA for _ in range(NBUF)],
        ),
        compiler_params=pltpu.CompilerParams(use_tc_tiling_on_sc=False),
    )
    def embed(idx_hbm, table_hbm, out_hbm, idx_v, rows, gsem, wsem):
        wid = lax.axis_index("s") * NC + lax.axis_index("c")
        base = wid * rows_per_w
        pltpu.sync_copy(idx_hbm.at[wid], idx_v)

        for jj in range(K):
            pltpu.async_copy(table_hbm.at[idx_v.at[jj]], rows[jj], gsem[jj])

        def step(i, _):
            for b in range(NBUF):
                j = i * NBUF + b
                jk = j + K
                bk = (b + K) % NBUF

                @pl.when(jnp.logical_and(jk >= NBUF, jk < rows_per_w))
                def _():
                    pltpu.make_async_copy(
                        rows[bk].at[pl.ds(0, hist)],
                        out_hbm.at[base, pl.ds(0, hist), pl.ds(0, D)],
                        wsem[bk],
                    ).wait()

                @pl.when(jk < rows_per_w)
                def _():
                    pltpu.async_copy(table_hbm.at[idx_v.at[jk]], rows[bk], gsem[bk])

                pltpu.make_async_copy(
                    table_hbm.at[idx_v.at[b]], rows[b], gsem[b]
                ).wait()
                pltpu.async_copy(
                    rows[b].at[pl.ds(0, hist)],
                    out_hbm.at[base + j, pl.ds(0, hist), pl.ds(0, D)],
                    wsem[b],
                )

            return 0

        lax.fori_loop(0, rows_per_w // NBUF, step, 0)

        for b in range(NBUF):
            pltpu.make_async_copy(
                rows[b].at[pl.ds(0, hist)],
                out_hbm.at[base, pl.ds(0, hist), pl.ds(0, D)],
                wsem[b],
            ).wait()

    return embed


def kernel(x, weight):
    b, h = x.shape
    xp = jnp.pad(x.astype(jnp.int32), ((0, 0), (0, PAD_H - h)))
    idx = xp.reshape(NW, b // NW, PAD_H)
    out = _make_embed_kernel(b, h)(idx, weight)
    return out[:, :h, : weight.shape[1]]


# speedup vs baseline: 2.4548x; 2.4548x over previous
import functools

import jax
import jax.numpy as jnp
from jax import lax
from jax.experimental import pallas as pl
from jax.experimental.pallas import tpu as pltpu
from jax.experimental.pallas import tpu_sc as plsc

NC = 2
NS = 16
NW = NC * NS
NBUF = 8
K = 4


def _make_embed_kernel(batch: int, hist: int, d: int):
    rows_per_w = batch // NW
    assert batch % NW == 0 and rows_per_w % NBUF == 0

    mesh = plsc.VectorSubcoreMesh(
        core_axis_name="c", subcore_axis_name="s",
        num_cores=NC, num_subcores=NS,
    )

    @functools.partial(
        pl.kernel,
        out_type=jax.ShapeDtypeStruct((batch, hist, d), jnp.float32),
        mesh=mesh,
        scratch_types=(
            pltpu.VMEM((rows_per_w, hist), jnp.int32),
            [pltpu.VMEM((hist, d), jnp.float32) for _ in range(NBUF)],
            [pltpu.SemaphoreType.DMA for _ in range(NBUF)],
            [pltpu.SemaphoreType.DMA for _ in range(NBUF)],
        ),
        compiler_params=pltpu.CompilerParams(use_tc_tiling_on_sc=False),
    )
    def embed(idx_hbm, table_hbm, out_hbm, idx_v, rows, gsem, wsem):
        wid = lax.axis_index("s") * NC + lax.axis_index("c")
        base = wid * rows_per_w
        pltpu.sync_copy(idx_hbm.at[pl.ds(base, rows_per_w)], idx_v)

        for jj in range(K):
            pltpu.async_copy(table_hbm.at[idx_v.at[jj]], rows[jj], gsem[jj])

        def step(i, _):
            for b in range(NBUF):
                j = i * NBUF + b
                jk = j + K
                bk = (b + K) % NBUF

                @pl.when(jnp.logical_and(jk >= NBUF, jk < rows_per_w))
                def _():
                    pltpu.make_async_copy(
                        rows[bk], out_hbm.at[base], wsem[bk]
                    ).wait()

                @pl.when(jk < rows_per_w)
                def _():
                    pltpu.async_copy(
                        table_hbm.at[idx_v.at[jk]], rows[bk], gsem[bk]
                    )

                pltpu.make_async_copy(
                    table_hbm.at[idx_v.at[b]], rows[b], gsem[b]
                ).wait()
                pltpu.async_copy(rows[b], out_hbm.at[base + j], wsem[b])

            return 0

        lax.fori_loop(0, rows_per_w // NBUF, step, 0)

        for b in range(NBUF):
            pltpu.make_async_copy(
                rows[b], out_hbm.at[base], wsem[b]
            ).wait()

    return embed


def kernel(x, weight):
    b, h = x.shape
    return _make_embed_kernel(b, h, weight.shape[1])(
        x.astype(jnp.int32), weight
    )
